# baseline (speedup 1.0000x reference)
import functools

import jax
import jax.numpy as jnp
from jax import lax
from jax.experimental import pallas as pl
from jax.experimental.pallas import tpu as pltpu, tpu_sc as plsc


def _make_sc_all(B, K, D, NC, NS):
    NW = NC * NS
    bw = B // NW
    CH = 128
    nch = bw * K // CH
    mesh = plsc.VectorSubcoreMesh(core_axis_name="c", subcore_axis_name="s")

    @functools.partial(
        pl.kernel,
        mesh=mesh,
        out_type=[
            jax.ShapeDtypeStruct((B, K), jnp.int32),
            jax.ShapeDtypeStruct((B, D), jnp.float32),
            jax.ShapeDtypeStruct((B, D), jnp.float32),
            jax.ShapeDtypeStruct((B * K, D), jnp.float32),
        ],
        scratch_types=[
            pltpu.VMEM((bw,), jnp.int32),
            pltpu.VMEM((bw,), jnp.int32),
            pltpu.VMEM((bw, K), jnp.int32),
            pltpu.VMEM((bw, K), jnp.int32),
            pltpu.VMEM((bw * K,), jnp.int32),
            pltpu.VMEM((bw, D), jnp.float32),
            pltpu.VMEM((bw, D), jnp.float32),
            pltpu.VMEM((CH, D), jnp.float32),
            pltpu.VMEM((CH, D), jnp.float32),
            pltpu.SemaphoreType.DMA,
            pltpu.SemaphoreType.DMA,
            pltpu.SemaphoreType.DMA,
            pltpu.SemaphoreType.DMA,
            pltpu.SemaphoreType.DMA,
            pltpu.SemaphoreType.DMA,
        ],
        compiler_params=pltpu.CompilerParams(use_tc_tiling_on_sc=False),
    )
    def sc_all(u_hbm, i_hbm, adj_e_hbm, adj_r_hbm, ent_hbm,
               rid_out, h_out, u_out, t_out,
               i_v, u_v, eid_v, rid_v, eflat_v, h_v, uu_v, tb0, tb1,
               s0, s1, s2, s3, g0, g1):
        wid = lax.axis_index("s") * NC + lax.axis_index("c")
        base = wid * bw
        pltpu.sync_copy(i_hbm.at[pl.ds(base, bw)], i_v)
        pltpu.sync_copy(u_hbm.at[pl.ds(base, bw)], u_v)
        c0 = pltpu.async_copy(adj_e_hbm.at[i_v], eid_v, s0)
        c1 = pltpu.async_copy(adj_r_hbm.at[i_v], rid_v, s1)
        c2 = pltpu.async_copy(ent_hbm.at[i_v], h_v, s2)
        c3 = pltpu.async_copy(ent_hbm.at[u_v], uu_v, s3)

        c0.wait()
        for b in range(bw):
            eflat_v[pl.ds(b * K, K)] = eid_v[b, :]

        bufs = (tb0, tb1)
        sems = (g0, g1)
        tbase = wid * (bw * K)
        prev = None
        for c in range(nch):
            bsel = c % 2
            d = pltpu.async_copy(
                ent_hbm.at[eflat_v.at[pl.ds(c * CH, CH)]], bufs[bsel],
                sems[bsel])
            if prev is not None:
                pd, pb, pc = prev
                pd.wait()
                pltpu.sync_copy(bufs[pb], t_out.at[pl.ds(tbase + pc * CH, CH)])
            prev = (d, bsel, c)

        c1.wait()
        pltpu.sync_copy(rid_v, rid_out.at[pl.ds(base, bw)])
        c2.wait()
        pltpu.sync_copy(h_v, h_out.at[pl.ds(base, bw)])
        c3.wait()
        pltpu.sync_copy(uu_v, u_out.at[pl.ds(base, bw)])

        pd, pb, pc = prev
        pd.wait()
        pltpu.sync_copy(bufs[pb], t_out.at[pl.ds(tbase + pc * CH, CH)])

    return sc_all


def _renorm(e):
    n2 = jnp.sum(e * e, axis=-1, keepdims=True)
    return e * jnp.where(n2 > 1.0, lax.rsqrt(n2), 1.0)


def _leaky(x):
    return jnp.where(x >= 0, x, 0.2 * x)


def _tc_body(K, rid_ref, h_ref, u_ref, t_ref, rel_ref,
             W1_ref, b1_ref, W2_ref, b2_ref,
             wl1_ref, wl1b_ref, wl2_ref, wl2b_ref, wl3_ref, wl3b_ref,
             out_ref):
    f32 = jnp.float32
    rel = _renorm(rel_ref[...])
    user = _renorm(u_ref[...])
    h = _renorm(h_ref[...])

    UR = jnp.dot(user, rel.T, preferred_element_type=f32)
    rid = rid_ref[...]
    ur = jnp.zeros(rid.shape, f32)
    for r in range(rel.shape[0]):
        ur = jnp.where(rid == r, UR[:, r:r + 1], ur)

    m = jnp.max(ur, axis=-1, keepdims=True)
    e = jnp.exp(ur - m)
    w = e / jnp.sum(e, axis=-1, keepdims=True)

    t = t_ref[...]
    n2 = jnp.sum(t * t, axis=2, keepdims=True)
    scale = w[:, :, None] * jnp.where(n2 > 1.0, lax.rsqrt(n2), 1.0)
    wt = scale * t
    s1 = jnp.sum(wt, axis=1)
    s2 = jnp.sum(wt * wt, axis=1)
    Nh = s1 * s1 - s2

    W1 = W1_ref[...]
    W2 = W2_ref[...]
    b1 = b1_ref[...]
    b2 = b2_ref[...]
    item = (_leaky(jnp.dot(h + Nh, W1, preferred_element_type=f32) + b1)
            + _leaky(jnp.dot(h * Nh, W2, preferred_element_type=f32) + b2))
    uo = (_leaky(jnp.dot(user + user, W1, preferred_element_type=f32) + b1)
          + _leaky(jnp.dot(user * user, W2, preferred_element_type=f32) + b2))

    D = h.shape[-1]
    wl1 = wl1_ref[...]
    l1 = (jnp.dot(uo, wl1[0:D], preferred_element_type=f32)
          + jnp.dot(item, wl1[D:2 * D], preferred_element_type=f32)
          + jnp.dot(uo + item, wl1[2 * D:3 * D], preferred_element_type=f32)
          + jnp.dot(uo * item, wl1[3 * D:4 * D], preferred_element_type=f32)
          + wl1b_ref[...])
    l2 = jnp.dot(l1, wl2_ref[...], preferred_element_type=f32) + wl2b_ref[...]
    l3 = jnp.dot(l2, wl3_ref[...], preferred_element_type=f32) + wl3b_ref[...]
    out_ref[...] = 1.0 / (1.0 + jnp.exp(-l3))


def _tc_compute(r_ids, h_rows, u_rows, t3, rel_table,
                W1_w, W1_b, W2_w, W2_b, wl1_w, wl1_b, wl2_w, wl2_b,
                wl3_w, wl3_b):
    B, K = r_ids.shape
    D = h_rows.shape[-1]
    NREL = rel_table.shape[0]
    bb = 512
    grid = (B // bb,)

    def full(shape):
        return pl.BlockSpec(shape, lambda b: (0,) * len(shape))

    out = pl.pallas_call(
        functools.partial(_tc_body, K),
        grid=grid,
        in_specs=[
            pl.BlockSpec((bb, K), lambda b: (b, 0)),
            pl.BlockSpec((bb, D), lambda b: (b, 0)),
            pl.BlockSpec((bb, D), lambda b: (b, 0)),
            pl.BlockSpec((bb, K, D), lambda b: (b, 0, 0)),
            full((NREL, D)),
            full((D, D)), full((D,)),
            full((D, D)), full((D,)),
            full((4 * D, D)), full((D,)),
            full((D, D // 2)), full((D // 2,)),
            full((D // 2, 1)), full((1,)),
        ],
        out_specs=pl.BlockSpec((bb, 1), lambda b: (b, 0)),
        out_shape=jax.ShapeDtypeStruct((B, 1), jnp.float32),
    )(r_ids, h_rows, u_rows, t3, rel_table,
      W1_w, W1_b, W2_w, W2_b, wl1_w, wl1_b, wl2_w, wl2_b, wl3_w, wl3_b)
    return out[:, 0]


def kernel(u, i, adj_entity, adj_relation, entity_table, relation_table,
           W1_w, W1_b, W2_w, W2_b, wl1_w, wl1_b, wl2_w, wl2_b, wl3_w, wl3_b):
    B = u.shape[0]
    N, K = adj_entity.shape
    D = entity_table.shape[1]
    info = plsc.get_sparse_core_info()
    NC, NS = info.num_cores, info.num_subcores

    H = B // 2
    sc = _make_sc_all(H, K, D, NC, NS)
    outs = []
    for p in range(2):
        sl = slice(p * H, (p + 1) * H)
        r_ids, h_rows, u_rows, t_rows = sc(
            u[sl], i[sl], adj_entity, adj_relation, entity_table)
        outs.append(_tc_compute(r_ids, h_rows, u_rows,
                                t_rows.reshape(H, K, D),
                                relation_table, W1_w, W1_b, W2_w, W2_b,
                                wl1_w, wl1_b, wl2_w, wl2_b, wl3_w, wl3_b))
    return jnp.concatenate(outs)


# speedup vs baseline: 1.0735x; 1.0735x over previous
import functools

import jax
import jax.numpy as jnp
from jax import lax
from jax.experimental import pallas as pl
from jax.experimental.pallas import tpu as pltpu, tpu_sc as plsc


def _make_sc_all(B, K, D, NC, NS):
    NW = NC * NS
    bw = B // NW
    CH = 128
    nch = bw * K // CH
    mesh = plsc.VectorSubcoreMesh(core_axis_name="c", subcore_axis_name="s")

    @functools.partial(
        pl.kernel,
        mesh=mesh,
        out_type=[
            jax.ShapeDtypeStruct((B, K), jnp.int32),
            jax.ShapeDtypeStruct((B, D), jnp.float32),
            jax.ShapeDtypeStruct((B, D), jnp.float32),
            jax.ShapeDtypeStruct((B * K, D), jnp.float32),
        ],
        scratch_types=[
            pltpu.VMEM((bw,), jnp.int32),
            pltpu.VMEM((bw,), jnp.int32),
            pltpu.VMEM((bw, K), jnp.int32),
            pltpu.VMEM((bw, K), jnp.int32),
            pltpu.VMEM((bw * K,), jnp.int32),
            pltpu.VMEM((bw, D), jnp.float32),
            pltpu.VMEM((bw, D), jnp.float32),
            pltpu.VMEM((CH, D), jnp.float32),
            pltpu.VMEM((CH, D), jnp.float32),
            pltpu.SemaphoreType.DMA,
            pltpu.SemaphoreType.DMA,
            pltpu.SemaphoreType.DMA,
            pltpu.SemaphoreType.DMA,
            pltpu.SemaphoreType.DMA,
            pltpu.SemaphoreType.DMA,
        ],
        compiler_params=pltpu.CompilerParams(use_tc_tiling_on_sc=False),
    )
    def sc_all(u_hbm, i_hbm, adj_e_hbm, adj_r_hbm, ent_hbm,
               rid_out, h_out, u_out, t_out,
               i_v, u_v, eid_v, rid_v, eflat_v, h_v, uu_v, tb0, tb1,
               s0, s1, s2, s3, g0, g1):
        wid = lax.axis_index("s") * NC + lax.axis_index("c")
        base = wid * bw
        pltpu.sync_copy(i_hbm.at[pl.ds(base, bw)], i_v)
        pltpu.sync_copy(u_hbm.at[pl.ds(base, bw)], u_v)
        c0 = pltpu.async_copy(adj_e_hbm.at[i_v], eid_v, s0)
        c1 = pltpu.async_copy(adj_r_hbm.at[i_v], rid_v, s1)
        c2 = pltpu.async_copy(ent_hbm.at[i_v], h_v, s2)
        c3 = pltpu.async_copy(ent_hbm.at[u_v], uu_v, s3)

        c0.wait()
        for b in range(bw):
            eflat_v[pl.ds(b * K, K)] = eid_v[b, :]

        bufs = (tb0, tb1)
        sems = (g0, g1)
        tbase = wid * (bw * K)
        prev = None
        for c in range(nch):
            bsel = c % 2
            d = pltpu.async_copy(
                ent_hbm.at[eflat_v.at[pl.ds(c * CH, CH)]], bufs[bsel],
                sems[bsel])
            if prev is not None:
                pd, pb, pc = prev
                pd.wait()
                pltpu.sync_copy(bufs[pb], t_out.at[pl.ds(tbase + pc * CH, CH)])
            prev = (d, bsel, c)

        c1.wait()
        pltpu.sync_copy(rid_v, rid_out.at[pl.ds(base, bw)])
        c2.wait()
        pltpu.sync_copy(h_v, h_out.at[pl.ds(base, bw)])
        c3.wait()
        pltpu.sync_copy(uu_v, u_out.at[pl.ds(base, bw)])

        pd, pb, pc = prev
        pd.wait()
        pltpu.sync_copy(bufs[pb], t_out.at[pl.ds(tbase + pc * CH, CH)])

    return sc_all


def _renorm(e):
    n2 = jnp.sum(e * e, axis=-1, keepdims=True)
    return e * jnp.where(n2 > 1.0, lax.rsqrt(n2), 1.0)


def _leaky(x):
    return jnp.where(x >= 0, x, 0.2 * x)


def _tc_body(K, rid_ref, h_ref, u_ref, t_ref, rel_ref,
             W1_ref, b1_ref, W2_ref, b2_ref,
             wl1_ref, wl1b_ref, wl2_ref, wl2b_ref, wl3_ref, wl3b_ref,
             out_ref):
    f32 = jnp.float32
    rel = _renorm(rel_ref[...])
    user = _renorm(u_ref[...])
    h = _renorm(h_ref[...])

    UR = jnp.dot(user, rel.T, preferred_element_type=f32)
    rid = rid_ref[...]
    ur = jnp.zeros(rid.shape, f32)
    for r in range(rel.shape[0]):
        ur = jnp.where(rid == r, UR[:, r:r + 1], ur)

    m = jnp.max(ur, axis=-1, keepdims=True)
    e = jnp.exp(ur - m)
    w = e / jnp.sum(e, axis=-1, keepdims=True)

    t = t_ref[...]
    n2 = jnp.sum(t * t, axis=2, keepdims=True)
    scale = w[:, :, None] * jnp.where(n2 > 1.0, lax.rsqrt(n2), 1.0)
    wt = scale * t
    s1 = jnp.sum(wt, axis=1)
    s2 = jnp.sum(wt * wt, axis=1)
    Nh = s1 * s1 - s2

    W1 = W1_ref[...]
    W2 = W2_ref[...]
    b1 = b1_ref[...]
    b2 = b2_ref[...]
    item = (_leaky(jnp.dot(h + Nh, W1, preferred_element_type=f32) + b1)
            + _leaky(jnp.dot(h * Nh, W2, preferred_element_type=f32) + b2))
    uo = (_leaky(jnp.dot(user + user, W1, preferred_element_type=f32) + b1)
          + _leaky(jnp.dot(user * user, W2, preferred_element_type=f32) + b2))

    D = h.shape[-1]
    wl1 = wl1_ref[...]
    l1 = (jnp.dot(uo, wl1[0:D], preferred_element_type=f32)
          + jnp.dot(item, wl1[D:2 * D], preferred_element_type=f32)
          + jnp.dot(uo + item, wl1[2 * D:3 * D], preferred_element_type=f32)
          + jnp.dot(uo * item, wl1[3 * D:4 * D], preferred_element_type=f32)
          + wl1b_ref[...])
    l2 = jnp.dot(l1, wl2_ref[...], preferred_element_type=f32) + wl2b_ref[...]
    l3 = jnp.dot(l2, wl3_ref[...], preferred_element_type=f32) + wl3b_ref[...]
    out_ref[...] = 1.0 / (1.0 + jnp.exp(-l3))


def _tc_compute(r_ids, h_rows, u_rows, t3, rel_table,
                W1_w, W1_b, W2_w, W2_b, wl1_w, wl1_b, wl2_w, wl2_b,
                wl3_w, wl3_b):
    B, K = r_ids.shape
    D = h_rows.shape[-1]
    NREL = rel_table.shape[0]
    bb = 512
    grid = (B // bb,)

    def full(shape):
        return pl.BlockSpec(shape, lambda b: (0,) * len(shape))

    out = pl.pallas_call(
        functools.partial(_tc_body, K),
        grid=grid,
        in_specs=[
            pl.BlockSpec((bb, K), lambda b: (b, 0)),
            pl.BlockSpec((bb, D), lambda b: (b, 0)),
            pl.BlockSpec((bb, D), lambda b: (b, 0)),
            pl.BlockSpec((bb, K, D), lambda b: (b, 0, 0)),
            full((NREL, D)),
            full((D, D)), full((D,)),
            full((D, D)), full((D,)),
            full((4 * D, D)), full((D,)),
            full((D, D // 2)), full((D // 2,)),
            full((D // 2, 1)), full((1,)),
        ],
        out_specs=pl.BlockSpec((bb, 1), lambda b: (b, 0)),
        out_shape=jax.ShapeDtypeStruct((B, 1), jnp.float32),
    )(r_ids, h_rows, u_rows, t3, rel_table,
      W1_w, W1_b, W2_w, W2_b, wl1_w, wl1_b, wl2_w, wl2_b, wl3_w, wl3_b)
    return out[:, 0]


def kernel(u, i, adj_entity, adj_relation, entity_table, relation_table,
           W1_w, W1_b, W2_w, W2_b, wl1_w, wl1_b, wl2_w, wl2_b, wl3_w, wl3_b):
    B = u.shape[0]
    N, K = adj_entity.shape
    D = entity_table.shape[1]
    info = plsc.get_sparse_core_info()
    NC, NS = info.num_cores, info.num_subcores

    af_e = lax.optimization_barrier(adj_entity.reshape(N * K))
    af_r = lax.optimization_barrier(adj_relation.reshape(N * K))
    r_ids, h_rows, u_rows, t_rows = _make_sc_all(B, K, D, NC, NS)(
        u, i, af_e.reshape(N, K), af_r.reshape(N, K), entity_table)
    return _tc_compute(r_ids, h_rows, u_rows, t_rows.reshape(B, K, D),
                       relation_table, W1_w, W1_b, W2_w, W2_b,
                       wl1_w, wl1_b, wl2_w, wl2_b, wl3_w, wl3_b)


# speedup vs baseline: 1.5521x; 1.4458x over previous
import functools

import jax
import jax.numpy as jnp
from jax import lax
from jax.experimental import pallas as pl
from jax.experimental.pallas import tpu as pltpu, tpu_sc as plsc


def _make_sc_all(B, N, K, D, NC, NS):
    NW = NC * NS
    bw = B // NW
    CH = 128
    nch = bw * K // CH
    NG = bw // 16
    mesh = plsc.VectorSubcoreMesh(core_axis_name="c", subcore_axis_name="s")

    @functools.partial(
        pl.kernel,
        mesh=mesh,
        out_type=[
            jax.ShapeDtypeStruct((K * B,), jnp.int32),
            jax.ShapeDtypeStruct((B, D), jnp.float32),
            jax.ShapeDtypeStruct((B, D), jnp.float32),
            jax.ShapeDtypeStruct((K * B, D), jnp.float32),
        ],
        scratch_types=[
            pltpu.VMEM((bw,), jnp.int32),
            pltpu.VMEM((bw,), jnp.int32),
            pltpu.VMEM((bw * K,), jnp.int32),
            pltpu.VMEM((CH,), jnp.int32),
            pltpu.VMEM((CH,), jnp.int32),
            pltpu.VMEM((CH,), jnp.int32),
            pltpu.VMEM((CH,), jnp.int32),
            pltpu.VMEM((bw, D), jnp.float32),
            pltpu.VMEM((bw, D), jnp.float32),
            pltpu.VMEM((CH, D), jnp.float32),
            pltpu.VMEM((CH, D), jnp.float32),
            pltpu.SemaphoreType.DMA,
            pltpu.SemaphoreType.DMA,
            pltpu.SemaphoreType.DMA,
            pltpu.SemaphoreType.DMA,
            pltpu.SemaphoreType.DMA,
            pltpu.SemaphoreType.DMA,
            pltpu.SemaphoreType.DMA,
            pltpu.SemaphoreType.DMA,
        ],
    )
    def sc_all(u_hbm, i_hbm, ate_hbm, atr_hbm, ent_hbm,
               rid_out, h_out, u_out, t_out,
               i_v, u_v, pos_v, eb0, eb1, rb0, rb1, h_v, uu_v, tb0, tb1,
               sh, su, se0, se1, sr0, sr1, g0, g1):
        wid = lax.axis_index("s") * NC + lax.axis_index("c")
        base = wid * bw
        pltpu.sync_copy(i_hbm.at[pl.ds(base, bw)], i_v)
        pltpu.sync_copy(u_hbm.at[pl.ds(base, bw)], u_v)
        ch = pltpu.async_copy(ent_hbm.at[i_v], h_v, sh)
        cu = pltpu.async_copy(ent_hbm.at[u_v], uu_v, su)

        for g in range(NG):
            ig = i_v[pl.ds(g * 16, 16)]
            for k in range(K):
                pos_v[pl.ds(k * bw + g * 16, 16)] = ig + (k * N)

        ebufs = (eb0, eb1)
        rbufs = (rb0, rb1)
        esems = (se0, se1)
        rsems = (sr0, sr1)
        tbufs = (tb0, tb1)
        tsems = (g0, g1)

        ecur = pltpu.async_copy(ate_hbm.at[pos_v.at[pl.ds(0, CH)]], eb0, se0)
        rcur = pltpu.async_copy(atr_hbm.at[pos_v.at[pl.ds(0, CH)]], rb0, sr0)
        prev = None
        for c in range(nch):
            b = c % 2
            nb = (c + 1) % 2
            ecur.wait()
            rcur.wait()
            tcur = pltpu.async_copy(ent_hbm.at[ebufs[b]], tbufs[b], tsems[b])
            if prev is not None:
                pt, pb, pc = prev
                pt.wait()
                pltpu.sync_copy(tbufs[pb], t_out.at[pl.ds(pc * B + base, CH)])
            if c + 1 < nch:
                sl = pos_v.at[pl.ds((c + 1) * CH, CH)]
                ecur = pltpu.async_copy(ate_hbm.at[sl], ebufs[nb], esems[nb])
                rcur = pltpu.async_copy(atr_hbm.at[sl], rbufs[nb], rsems[nb])
            pltpu.sync_copy(rbufs[b], rid_out.at[pl.ds(c * B + base, CH)])
            prev = (tcur, b, c)

        ch.wait()
        pltpu.sync_copy(h_v, h_out.at[pl.ds(base, bw)])
        cu.wait()
        pltpu.sync_copy(uu_v, u_out.at[pl.ds(base, bw)])

        pt, pb, pc = prev
        pt.wait()
        pltpu.sync_copy(tbufs[pb], t_out.at[pl.ds(pc * B + base, CH)])

    return sc_all


def _renorm(e):
    n2 = jnp.sum(e * e, axis=-1, keepdims=True)
    return e * jnp.where(n2 > 1.0, lax.rsqrt(n2), 1.0)


def _leaky(x):
    return jnp.where(x >= 0, x, 0.2 * x)


def _tc_body(K, rid_ref, h_ref, u_ref, t_ref, rel_ref,
             W1_ref, b1_ref, W2_ref, b2_ref,
             wl1_ref, wl1b_ref, wl2_ref, wl2b_ref, wl3_ref, wl3b_ref,
             out_ref):
    f32 = jnp.float32
    rel = _renorm(rel_ref[...])
    user = _renorm(u_ref[...])
    h = _renorm(h_ref[...])

    UR = jnp.dot(user, rel.T, preferred_element_type=f32)
    NREL = rel.shape[0]
    rid = rid_ref[...]
    ur = jnp.zeros(rid.shape, f32)
    for r in range(NREL):
        ur = jnp.where(rid == r, UR[:, r][None, :], ur)

    m = jnp.max(ur, axis=0, keepdims=True)
    e = jnp.exp(ur - m)
    w = e / jnp.sum(e, axis=0, keepdims=True)

    t = t_ref[...]
    n2 = jnp.sum(t * t, axis=2, keepdims=True)
    scale = w[:, :, None] * jnp.where(n2 > 1.0, lax.rsqrt(n2), 1.0)
    wt = scale * t
    s1 = jnp.sum(wt, axis=0)
    s2 = jnp.sum(wt * wt, axis=0)
    Nh = s1 * s1 - s2

    W1 = W1_ref[...]
    W2 = W2_ref[...]
    b1 = b1_ref[...]
    b2 = b2_ref[...]
    item = (_leaky(jnp.dot(h + Nh, W1, preferred_element_type=f32) + b1)
            + _leaky(jnp.dot(h * Nh, W2, preferred_element_type=f32) + b2))
    uo = (_leaky(jnp.dot(user + user, W1, preferred_element_type=f32) + b1)
          + _leaky(jnp.dot(user * user, W2, preferred_element_type=f32) + b2))

    D = h.shape[-1]
    wl1 = wl1_ref[...]
    l1 = (jnp.dot(uo, wl1[0:D], preferred_element_type=f32)
          + jnp.dot(item, wl1[D:2 * D], preferred_element_type=f32)
          + jnp.dot(uo + item, wl1[2 * D:3 * D], preferred_element_type=f32)
          + jnp.dot(uo * item, wl1[3 * D:4 * D], preferred_element_type=f32)
          + wl1b_ref[...])
    l2 = jnp.dot(l1, wl2_ref[...], preferred_element_type=f32) + wl2b_ref[...]
    l3 = jnp.dot(l2, wl3_ref[...], preferred_element_type=f32) + wl3b_ref[...]
    out_ref[...] = 1.0 / (1.0 + jnp.exp(-l3))


def _tc_compute(r_ids, h_rows, u_rows, t3, rel_table,
                W1_w, W1_b, W2_w, W2_b, wl1_w, wl1_b, wl2_w, wl2_b,
                wl3_w, wl3_b):
    K, B = r_ids.shape
    D = h_rows.shape[-1]
    NREL = rel_table.shape[0]
    bb = 512
    grid = (B // bb,)

    def full(shape):
        return pl.BlockSpec(shape, lambda b: (0,) * len(shape))

    out = pl.pallas_call(
        functools.partial(_tc_body, K),
        grid=grid,
        in_specs=[
            pl.BlockSpec((K, bb), lambda b: (0, b)),
            pl.BlockSpec((bb, D), lambda b: (b, 0)),
            pl.BlockSpec((bb, D), lambda b: (b, 0)),
            pl.BlockSpec((K, bb, D), lambda b: (0, b, 0)),
            full((NREL, D)),
            full((D, D)), full((D,)),
            full((D, D)), full((D,)),
            full((4 * D, D)), full((D,)),
            full((D, D // 2)), full((D // 2,)),
            full((D // 2, 1)), full((1,)),
        ],
        out_specs=pl.BlockSpec((bb, 1), lambda b: (b, 0)),
        out_shape=jax.ShapeDtypeStruct((B, 1), jnp.float32),
    )(r_ids, h_rows, u_rows, t3, rel_table,
      W1_w, W1_b, W2_w, W2_b, wl1_w, wl1_b, wl2_w, wl2_b, wl3_w, wl3_b)
    return out[:, 0]


def kernel(u, i, adj_entity, adj_relation, entity_table, relation_table,
           W1_w, W1_b, W2_w, W2_b, wl1_w, wl1_b, wl2_w, wl2_b, wl3_w, wl3_b):
    B = u.shape[0]
    N, K = adj_entity.shape
    D = entity_table.shape[1]
    info = plsc.get_sparse_core_info()
    NC, NS = info.num_cores, info.num_subcores

    ate = adj_entity.T.reshape(N * K)
    atr = adj_relation.T.reshape(N * K)
    r_ids, h_rows, u_rows, t_rows = _make_sc_all(B, N, K, D, NC, NS)(
        u, i, ate, atr, entity_table)
    return _tc_compute(r_ids.reshape(K, B), h_rows, u_rows,
                       t_rows.reshape(K, B, D),
                       relation_table, W1_w, W1_b, W2_w, W2_b,
                       wl1_w, wl1_b, wl2_w, wl2_b, wl3_w, wl3_b)


# speedup vs baseline: 1.6218x; 1.0449x over previous
import functools

import jax
import jax.numpy as jnp
from jax import lax
from jax.experimental import pallas as pl
from jax.experimental.pallas import tpu as pltpu, tpu_sc as plsc


def _make_sc_all(B, N, K, D, NC, NS):
    NW = NC * NS
    bw = B // NW
    CH = 128
    nch = bw * K // CH
    NG = bw // 16
    mesh = plsc.VectorSubcoreMesh(core_axis_name="c", subcore_axis_name="s")

    @functools.partial(
        pl.kernel,
        mesh=mesh,
        out_type=[
            jax.ShapeDtypeStruct((K * B,), jnp.int32),
            jax.ShapeDtypeStruct((B, D), jnp.float32),
            jax.ShapeDtypeStruct((B, D), jnp.float32),
            jax.ShapeDtypeStruct((K * B, D), jnp.float32),
        ],
        scratch_types=[
            pltpu.VMEM((bw,), jnp.int32),
            pltpu.VMEM((bw,), jnp.int32),
            pltpu.VMEM((bw * K,), jnp.int32),
            pltpu.VMEM((bw * K,), jnp.int32),
            pltpu.VMEM((bw * K,), jnp.int32),
            pltpu.VMEM((bw, D), jnp.float32),
            pltpu.VMEM((bw, D), jnp.float32),
            pltpu.VMEM((CH, D), jnp.float32),
            pltpu.VMEM((CH, D), jnp.float32),
            pltpu.SemaphoreType.DMA,
            pltpu.SemaphoreType.DMA,
            pltpu.SemaphoreType.DMA,
            pltpu.SemaphoreType.DMA,
            pltpu.SemaphoreType.DMA,
            pltpu.SemaphoreType.DMA,
            pltpu.SemaphoreType.DMA,
            pltpu.SemaphoreType.DMA,
        ],
    )
    def sc_all(u_hbm, i_hbm, ate_hbm, atr_hbm, ent_hbm,
               rid_out, h_out, u_out, t_out,
               i_v, u_v, pos_v, eids_v, rids_v, h_v, uu_v, tb0, tb1,
               sh, su, se0, se1, sr0, sr1, g0, g1):
        wid = lax.axis_index("s") * NC + lax.axis_index("c")
        base = wid * bw
        pltpu.sync_copy(i_hbm.at[pl.ds(base, bw)], i_v)
        pltpu.sync_copy(u_hbm.at[pl.ds(base, bw)], u_v)
        ch = pltpu.async_copy(ent_hbm.at[i_v], h_v, sh)
        cu = pltpu.async_copy(ent_hbm.at[u_v], uu_v, su)

        for g in range(NG):
            ig = i_v[pl.ds(g * 16, 16)]
            for k in range(K):
                pos_v[pl.ds(k * bw + g * 16, 16)] = ig + (k * N)

        tbufs = (tb0, tb1)
        tsems = (g0, g1)

        ecps = []
        rcps = []
        for c in range(nch):
            sl = pos_v.at[pl.ds(c * CH, CH)]
            ecps.append(pltpu.async_copy(
                ate_hbm.at[sl], eids_v.at[pl.ds(c * CH, CH)], se0))
            rcps.append(pltpu.async_copy(
                atr_hbm.at[sl], rids_v.at[pl.ds(c * CH, CH)], sr0))
        for cp in ecps:
            cp.wait()

        prev = None
        for c in range(nch):
            b = c % 2
            tcur = pltpu.async_copy(
                ent_hbm.at[eids_v.at[pl.ds(c * CH, CH)]], tbufs[b], tsems[b])
            if prev is not None:
                pt, pb, pc = prev
                pt.wait()
                pltpu.sync_copy(tbufs[pb], t_out.at[pl.ds(pc * B + base, CH)])
            prev = (tcur, b, c)

        for cp in rcps:
            cp.wait()
        for c in range(nch):
            pltpu.sync_copy(rids_v.at[pl.ds(c * CH, CH)],
                            rid_out.at[pl.ds(c * B + base, CH)])

        ch.wait()
        pltpu.sync_copy(h_v, h_out.at[pl.ds(base, bw)])
        cu.wait()
        pltpu.sync_copy(uu_v, u_out.at[pl.ds(base, bw)])

        pt, pb, pc = prev
        pt.wait()
        pltpu.sync_copy(tbufs[pb], t_out.at[pl.ds(pc * B + base, CH)])

    return sc_all


def _renorm(e):
    n2 = jnp.sum(e * e, axis=-1, keepdims=True)
    return e * jnp.where(n2 > 1.0, lax.rsqrt(n2), 1.0)


def _leaky(x):
    return jnp.where(x >= 0, x, 0.2 * x)


def _tc_body(K, rid_ref, h_ref, u_ref, t_ref, rel_ref,
             W1_ref, b1_ref, W2_ref, b2_ref,
             wl1_ref, wl1b_ref, wl2_ref, wl2b_ref, wl3_ref, wl3b_ref,
             out_ref):
    f32 = jnp.float32
    rel = _renorm(rel_ref[...])
    user = _renorm(u_ref[...])
    h = _renorm(h_ref[...])

    UR = jnp.dot(user, rel.T, preferred_element_type=f32)
    NREL = rel.shape[0]
    rid = rid_ref[...]
    ur = jnp.zeros(rid.shape, f32)
    for r in range(NREL):
        ur = jnp.where(rid == r, UR[:, r][None, :], ur)

    m = jnp.max(ur, axis=0, keepdims=True)
    e = jnp.exp(ur - m)
    w = e / jnp.sum(e, axis=0, keepdims=True)

    t = t_ref[...]
    n2 = jnp.sum(t * t, axis=2, keepdims=True)
    scale = w[:, :, None] * jnp.where(n2 > 1.0, lax.rsqrt(n2), 1.0)
    wt = scale * t
    s1 = jnp.sum(wt, axis=0)
    s2 = jnp.sum(wt * wt, axis=0)
    Nh = s1 * s1 - s2

    W1 = W1_ref[...]
    W2 = W2_ref[...]
    b1 = b1_ref[...]
    b2 = b2_ref[...]
    item = (_leaky(jnp.dot(h + Nh, W1, preferred_element_type=f32) + b1)
            + _leaky(jnp.dot(h * Nh, W2, preferred_element_type=f32) + b2))
    uo = (_leaky(jnp.dot(user + user, W1, preferred_element_type=f32) + b1)
          + _leaky(jnp.dot(user * user, W2, preferred_element_type=f32) + b2))

    D = h.shape[-1]
    wl1 = wl1_ref[...]
    l1 = (jnp.dot(uo, wl1[0:D], preferred_element_type=f32)
          + jnp.dot(item, wl1[D:2 * D], preferred_element_type=f32)
          + jnp.dot(uo + item, wl1[2 * D:3 * D], preferred_element_type=f32)
          + jnp.dot(uo * item, wl1[3 * D:4 * D], preferred_element_type=f32)
          + wl1b_ref[...])
    l2 = jnp.dot(l1, wl2_ref[...], preferred_element_type=f32) + wl2b_ref[...]
    l3 = jnp.dot(l2, wl3_ref[...], preferred_element_type=f32) + wl3b_ref[...]
    out_ref[...] = 1.0 / (1.0 + jnp.exp(-l3))


def _tc_compute(r_ids, h_rows, u_rows, t3, rel_table,
                W1_w, W1_b, W2_w, W2_b, wl1_w, wl1_b, wl2_w, wl2_b,
                wl3_w, wl3_b):
    K, B = r_ids.shape
    D = h_rows.shape[-1]
    NREL = rel_table.shape[0]
    bb = 512
    grid = (B // bb,)

    def full(shape):
        return pl.BlockSpec(shape, lambda b: (0,) * len(shape))

    out = pl.pallas_call(
        functools.partial(_tc_body, K),
        grid=grid,
        in_specs=[
            pl.BlockSpec((K, bb), lambda b: (0, b)),
            pl.BlockSpec((bb, D), lambda b: (b, 0)),
            pl.BlockSpec((bb, D), lambda b: (b, 0)),
            pl.BlockSpec((K, bb, D), lambda b: (0, b, 0)),
            full((NREL, D)),
            full((D, D)), full((D,)),
            full((D, D)), full((D,)),
            full((4 * D, D)), full((D,)),
            full((D, D // 2)), full((D // 2,)),
            full((D // 2, 1)), full((1,)),
        ],
        out_specs=pl.BlockSpec((bb, 1), lambda b: (b, 0)),
        out_shape=jax.ShapeDtypeStruct((B, 1), jnp.float32),
    )(r_ids, h_rows, u_rows, t3, rel_table,
      W1_w, W1_b, W2_w, W2_b, wl1_w, wl1_b, wl2_w, wl2_b, wl3_w, wl3_b)
    return out[:, 0]


def kernel(u, i, adj_entity, adj_relation, entity_table, relation_table,
           W1_w, W1_b, W2_w, W2_b, wl1_w, wl1_b, wl2_w, wl2_b, wl3_w, wl3_b):
    B = u.shape[0]
    N, K = adj_entity.shape
    D = entity_table.shape[1]
    info = plsc.get_sparse_core_info()
    NC, NS = info.num_cores, info.num_subcores

    ate = adj_entity.T.reshape(N * K)
    atr = adj_relation.T.reshape(N * K)
    r_ids, h_rows, u_rows, t_rows = _make_sc_all(B, N, K, D, NC, NS)(
        u, i, ate, atr, entity_table)
    return _tc_compute(r_ids.reshape(K, B), h_rows, u_rows,
                       t_rows.reshape(K, B, D),
                       relation_table, W1_w, W1_b, W2_w, W2_b,
                       wl1_w, wl1_b, wl2_w, wl2_b, wl3_w, wl3_b)


# speedup vs baseline: 1.6631x; 1.0255x over previous
import functools

import jax
import jax.numpy as jnp
from jax import lax
from jax.experimental import pallas as pl
from jax.experimental.pallas import tpu as pltpu, tpu_sc as plsc


def _make_sc_all(B, N, K, D, NC, NS):
    NW = NC * NS
    bw = B // NW
    CH = 128
    nch = bw * K // CH
    NG = bw // 16
    mesh = plsc.VectorSubcoreMesh(core_axis_name="c", subcore_axis_name="s")

    @functools.partial(
        pl.kernel,
        mesh=mesh,
        out_type=[
            jax.ShapeDtypeStruct((K * B,), jnp.int32),
            jax.ShapeDtypeStruct((B, D), jnp.float32),
            jax.ShapeDtypeStruct((B, D), jnp.float32),
            jax.ShapeDtypeStruct((K * B, D), jnp.float32),
        ],
        scratch_types=[
            pltpu.VMEM((bw,), jnp.int32),
            pltpu.VMEM((bw,), jnp.int32),
            pltpu.VMEM((bw * K,), jnp.int32),
            pltpu.VMEM((bw * K,), jnp.int32),
            pltpu.VMEM((bw * K,), jnp.int32),
            pltpu.VMEM((bw, D), jnp.float32),
            pltpu.VMEM((bw, D), jnp.float32),
            pltpu.VMEM((CH, D), jnp.float32),
            pltpu.VMEM((CH, D), jnp.float32),
            pltpu.VMEM((CH, D), jnp.float32),
            pltpu.VMEM((CH, D), jnp.float32),
            pltpu.SemaphoreType.DMA,
            pltpu.SemaphoreType.DMA,
            pltpu.SemaphoreType.DMA,
            pltpu.SemaphoreType.DMA,
            pltpu.SemaphoreType.DMA,
            pltpu.SemaphoreType.DMA,
            pltpu.SemaphoreType.DMA,
            pltpu.SemaphoreType.DMA,
            pltpu.SemaphoreType.DMA,
            pltpu.SemaphoreType.DMA,
        ],
    )
    def sc_all(u_hbm, i_hbm, ate_hbm, atr_hbm, ent_hbm,
               rid_out, h_out, u_out, t_out,
               i_v, u_v, pos_v, eids_v, rids_v, h_v, uu_v,
               tb0, tb1, tb2, tb3,
               sh, su, se0, se1, sr0, sr1, g0, g1, g2, g3):
        wid = lax.axis_index("s") * NC + lax.axis_index("c")
        base = wid * bw
        pltpu.sync_copy(i_hbm.at[pl.ds(base, bw)], i_v)
        pltpu.sync_copy(u_hbm.at[pl.ds(base, bw)], u_v)
        ch = pltpu.async_copy(ent_hbm.at[i_v], h_v, sh)
        cu = pltpu.async_copy(ent_hbm.at[u_v], uu_v, su)

        for g in range(NG):
            ig = i_v[pl.ds(g * 16, 16)]
            for k in range(K):
                pos_v[pl.ds(k * bw + g * 16, 16)] = ig + (k * N)

        tbufs = (tb0, tb1, tb2, tb3)
        tsems = (g0, g1, g2, g3)
        NB = 4

        ecps = []
        rcps = []
        for c in range(nch):
            sl = pos_v.at[pl.ds(c * CH, CH)]
            ecps.append(pltpu.async_copy(
                ate_hbm.at[sl], eids_v.at[pl.ds(c * CH, CH)], se0))
            rcps.append(pltpu.async_copy(
                atr_hbm.at[sl], rids_v.at[pl.ds(c * CH, CH)], sr0))
        for cp in ecps:
            cp.wait()

        inflight = []
        for c in range(nch):
            b = c % NB
            if len(inflight) == NB:
                pt, pb, pc = inflight.pop(0)
                pt.wait()
                pltpu.sync_copy(tbufs[pb], t_out.at[pl.ds(pc * B + base, CH)])
            tcur = pltpu.async_copy(
                ent_hbm.at[eids_v.at[pl.ds(c * CH, CH)]], tbufs[b], tsems[b])
            inflight.append((tcur, b, c))

        for cp in rcps:
            cp.wait()
        for c in range(nch):
            pltpu.sync_copy(rids_v.at[pl.ds(c * CH, CH)],
                            rid_out.at[pl.ds(c * B + base, CH)])

        ch.wait()
        pltpu.sync_copy(h_v, h_out.at[pl.ds(base, bw)])
        cu.wait()
        pltpu.sync_copy(uu_v, u_out.at[pl.ds(base, bw)])

        for pt, pb, pc in inflight:
            pt.wait()
            pltpu.sync_copy(tbufs[pb], t_out.at[pl.ds(pc * B + base, CH)])

    return sc_all


def _renorm(e):
    n2 = jnp.sum(e * e, axis=-1, keepdims=True)
    return e * jnp.where(n2 > 1.0, lax.rsqrt(n2), 1.0)


def _leaky(x):
    return jnp.where(x >= 0, x, 0.2 * x)


def _tc_body(K, rid_ref, h_ref, u_ref, t_ref, rel_ref,
             W1_ref, b1_ref, W2_ref, b2_ref,
             wl1_ref, wl1b_ref, wl2_ref, wl2b_ref, wl3_ref, wl3b_ref,
             out_ref):
    f32 = jnp.float32
    rel = _renorm(rel_ref[...])
    user = _renorm(u_ref[...])
    h = _renorm(h_ref[...])

    UR = jnp.dot(user, rel.T, preferred_element_type=f32)
    NREL = rel.shape[0]
    rid = rid_ref[...]
    ur = jnp.zeros(rid.shape, f32)
    for r in range(NREL):
        ur = jnp.where(rid == r, UR[:, r][None, :], ur)

    m = jnp.max(ur, axis=0, keepdims=True)
    e = jnp.exp(ur - m)
    w = e / jnp.sum(e, axis=0, keepdims=True)

    t = t_ref[...]
    n2 = jnp.sum(t * t, axis=2, keepdims=True)
    scale = w[:, :, None] * jnp.where(n2 > 1.0, lax.rsqrt(n2), 1.0)
    wt = scale * t
    s1 = jnp.sum(wt, axis=0)
    s2 = jnp.sum(wt * wt, axis=0)
    Nh = s1 * s1 - s2

    W1 = W1_ref[...]
    W2 = W2_ref[...]
    b1 = b1_ref[...]
    b2 = b2_ref[...]
    item = (_leaky(jnp.dot(h + Nh, W1, preferred_element_type=f32) + b1)
            + _leaky(jnp.dot(h * Nh, W2, preferred_element_type=f32) + b2))
    uo = (_leaky(jnp.dot(user + user, W1, preferred_element_type=f32) + b1)
          + _leaky(jnp.dot(user * user, W2, preferred_element_type=f32) + b2))

    D = h.shape[-1]
    wl1 = wl1_ref[...]
    l1 = (jnp.dot(uo, wl1[0:D], preferred_element_type=f32)
          + jnp.dot(item, wl1[D:2 * D], preferred_element_type=f32)
          + jnp.dot(uo + item, wl1[2 * D:3 * D], preferred_element_type=f32)
          + jnp.dot(uo * item, wl1[3 * D:4 * D], preferred_element_type=f32)
          + wl1b_ref[...])
    l2 = jnp.dot(l1, wl2_ref[...], preferred_element_type=f32) + wl2b_ref[...]
    l3 = jnp.dot(l2, wl3_ref[...], preferred_element_type=f32) + wl3b_ref[...]
    out_ref[...] = 1.0 / (1.0 + jnp.exp(-l3))


def _tc_compute(r_ids, h_rows, u_rows, t3, rel_table,
                W1_w, W1_b, W2_w, W2_b, wl1_w, wl1_b, wl2_w, wl2_b,
                wl3_w, wl3_b):
    K, B = r_ids.shape
    D = h_rows.shape[-1]
    NREL = rel_table.shape[0]
    bb = 512
    grid = (B // bb,)

    def full(shape):
        return pl.BlockSpec(shape, lambda b: (0,) * len(shape))

    out = pl.pallas_call(
        functools.partial(_tc_body, K),
        grid=grid,
        in_specs=[
            pl.BlockSpec((K, bb), lambda b: (0, b)),
            pl.BlockSpec((bb, D), lambda b: (b, 0)),
            pl.BlockSpec((bb, D), lambda b: (b, 0)),
            pl.BlockSpec((K, bb, D), lambda b: (0, b, 0)),
            full((NREL, D)),
            full((D, D)), full((D,)),
            full((D, D)), full((D,)),
            full((4 * D, D)), full((D,)),
            full((D, D // 2)), full((D // 2,)),
            full((D // 2, 1)), full((1,)),
        ],
        out_specs=pl.BlockSpec((bb, 1), lambda b: (b, 0)),
        out_shape=jax.ShapeDtypeStruct((B, 1), jnp.float32),
    )(r_ids, h_rows, u_rows, t3, rel_table,
      W1_w, W1_b, W2_w, W2_b, wl1_w, wl1_b, wl2_w, wl2_b, wl3_w, wl3_b)
    return out[:, 0]


def kernel(u, i, adj_entity, adj_relation, entity_table, relation_table,
           W1_w, W1_b, W2_w, W2_b, wl1_w, wl1_b, wl2_w, wl2_b, wl3_w, wl3_b):
    B = u.shape[0]
    N, K = adj_entity.shape
    D = entity_table.shape[1]
    info = plsc.get_sparse_core_info()
    NC, NS = info.num_cores, info.num_subcores

    ate = adj_entity.T.reshape(N * K)
    atr = adj_relation.T.reshape(N * K)
    r_ids, h_rows, u_rows, t_rows = _make_sc_all(B, N, K, D, NC, NS)(
        u, i, ate, atr, entity_table)
    return _tc_compute(r_ids.reshape(K, B), h_rows, u_rows,
                       t_rows.reshape(K, B, D),
                       relation_table, W1_w, W1_b, W2_w, W2_b,
                       wl1_w, wl1_b, wl2_w, wl2_b, wl3_w, wl3_b)
